# Initial kernel scaffold; baseline (speedup 1.0000x reference)
import jax
import jax.numpy as jnp
from jax.experimental import pallas as pl


def kernel(x, gate_w, gate_b, fc_w, fc_b, out_w, out_b):
    raise NotImplementedError("write your pallas kernel here")


# speedup vs baseline: 2.9631x; 2.9631x over previous
import functools

import jax
import jax.numpy as jnp
from jax import lax
from jax.experimental import pallas as pl
from jax.experimental.pallas import tpu as pltpu
from jax.experimental.pallas import tpu_sc as plsc

H = 768
F_DIM = 768
F2 = 2 * F_DIM
E = 8
T = 2048
BT = 256
NB = T // BT
STEPS = NB + E - 1
NC = 2
NS = 16
NW = NC * NS
CHUNK = T // NW


def _erf(v):
    p = 0.3275911
    a1, a2, a3, a4, a5 = (0.254829592, -0.284496736, 1.421413741,
                          -1.453152027, 1.061405429)
    sg = jnp.sign(v)
    av = jnp.abs(v)
    t = 1.0 / (1.0 + p * av)
    poly = ((((a5 * t + a4) * t + a3) * t + a2) * t + a1) * t
    return sg * (1.0 - poly * jnp.exp(-av * av))


def _gelu(v):
    return 0.5 * v * (1.0 + _erf(v * 0.7071067811865476))


def _router_body(gate_w_ref, gate_b_ref, x_ref, pos_ref, counts_ref,
                 loss_ref, cnt_scr, offs_scr):
    s = pl.program_id(0)

    @pl.when(s == 0)
    def _():
        cnt_scr[...] = jnp.zeros_like(cnt_scr)
        offs_scr[...] = jnp.zeros_like(offs_scr)

    @pl.when(s == NB)
    def _():
        def _sh(u, k):
            return jnp.concatenate(
                [jnp.zeros((1, k), jnp.float32), u[:, :E - k]], axis=1)

        t = cnt_scr[...]
        t = t + _sh(t, 1)
        t = t + _sh(t, 2)
        t = t + _sh(t, 4)
        offs_scr[...] = _sh(t, 1)
        cnt_scr[...] = jnp.zeros_like(cnt_scr)

    x = x_ref[...]
    raw = lax.dot_general(x, gate_w_ref[...], (((1,), (1,)), ((), ())),
                          preferred_element_type=jnp.float32)
    raw = raw + gate_b_ref[...]
    eids = lax.broadcasted_iota(jnp.int32, (BT, E), 1)
    m = jnp.max(raw, axis=1, keepdims=True)
    idx = jnp.min(jnp.where(raw == m, eids, E), axis=1)
    onehot = (eids == idx[:, None]).astype(jnp.bfloat16)

    rit = lax.broadcasted_iota(jnp.int32, (BT, BT), 0)
    cit = lax.broadcasted_iota(jnp.int32, (BT, BT), 1)
    ltri = (rit > cit).astype(jnp.bfloat16)
    within = lax.dot_general(ltri, onehot, (((1,), (0,)), ((), ())),
                             preferred_element_type=jnp.float32)
    rank_mat = within + cnt_scr[...]

    @pl.when(s >= NB)
    def _():
        pos_mat = rank_mat + offs_scr[...]
        pos = jnp.sum(jnp.where(eids == idx[:, None], pos_mat, 0.0), axis=1)
        pos_ref[...] = pos.astype(jnp.int32)[:, None]

    newcnt = cnt_scr[...] + jnp.sum(
        onehot.astype(jnp.float32), axis=0, keepdims=True)
    cnt_scr[...] = newcnt

    @pl.when(s == 2 * NB - 1)
    def _():
        counts_ref[...] = newcnt.astype(jnp.int32)
        usage = (newcnt > 0.0).astype(jnp.float32)
        loss_ref[...] = (jnp.sum((usage - 1.0 / E) ** 2) + 1e-8).reshape(1, 1)


def _router_call(x, gate_w, gate_b):
    return pl.pallas_call(
        _router_body,
        grid=(2 * NB,),
        in_specs=[
            pl.BlockSpec((E, H), lambda s: (0, 0)),
            pl.BlockSpec((1, E), lambda s: (0, 0)),
            pl.BlockSpec((BT, H), lambda s: (s % NB, 0)),
        ],
        out_specs=[
            pl.BlockSpec((BT, 1), lambda s: (s % NB, 0)),
            pl.BlockSpec((1, E), lambda s: (0, 0)),
            pl.BlockSpec((1, 1), lambda s: (0, 0)),
        ],
        out_shape=[
            jax.ShapeDtypeStruct((T, 1), jnp.int32),
            jax.ShapeDtypeStruct((1, E), jnp.int32),
            jax.ShapeDtypeStruct((1, 1), jnp.float32),
        ],
        scratch_shapes=[pltpu.VMEM((1, E), jnp.float32),
                        pltpu.VMEM((1, E), jnp.float32)],
    )(gate_w, gate_b.reshape(1, E), x)


@functools.cache
def _get_dispatch():
    mesh = plsc.VectorSubcoreMesh(core_axis_name="c", subcore_axis_name="s")

    @functools.partial(
        pl.kernel,
        mesh=mesh,
        out_type=jax.ShapeDtypeStruct((T, H), jnp.float32),
        scratch_types=[
            pltpu.VMEM((CHUNK,), jnp.int32),
            pltpu.VMEM((CHUNK, H), jnp.float32),
            pltpu.SemaphoreType.DMA,
        ],
    )
    def _dispatch(x_hbm, pos_hbm, xs_hbm, pos_v, x_v, sem):
        wid = lax.axis_index("s") * NC + lax.axis_index("c")
        base = wid * CHUNK
        pltpu.sync_copy(pos_hbm.at[pl.ds(base, CHUNK)], pos_v)
        pltpu.sync_copy(x_hbm.at[pl.ds(base, CHUNK)], x_v)
        pltpu.async_copy(x_v, xs_hbm.at[pos_v], sem).wait()

    return _dispatch


@functools.cache
def _get_combine():
    mesh = plsc.VectorSubcoreMesh(core_axis_name="c", subcore_axis_name="s")

    @functools.partial(
        pl.kernel,
        mesh=mesh,
        out_type=jax.ShapeDtypeStruct((T, H), jnp.float32),
        scratch_types=[
            pltpu.VMEM((CHUNK,), jnp.int32),
            pltpu.VMEM((CHUNK, H), jnp.float32),
            pltpu.SemaphoreType.DMA,
        ],
    )
    def _combine(ys_hbm, pos_hbm, out_hbm, pos_v, y_v, sem):
        wid = lax.axis_index("s") * NC + lax.axis_index("c")
        base = wid * CHUNK
        pltpu.sync_copy(pos_hbm.at[pl.ds(base, CHUNK)], pos_v)
        pltpu.async_copy(ys_hbm.at[pos_v], y_v, sem).wait()
        pltpu.sync_copy(y_v, out_hbm.at[pl.ds(base, CHUNK)])

    return _combine


def _ffn_body(bid_ref, eid_ref, first_ref, valid_ref, offs_ref,
              xs_ref, fcw_ref, fcb_ref, outw_ref, outb_ref, ys_ref):
    s = pl.program_id(0)
    e = eid_ref[s]
    b = bid_ref[s]
    x = xs_ref[...]
    h = lax.dot_general(x, fcw_ref[0], (((1,), (1,)), ((), ())),
                        preferred_element_type=jnp.float32)
    h = h + fcb_ref[0]
    g = h[:, :F_DIM] * _gelu(h[:, F_DIM:])
    eo = lax.dot_general(g, outw_ref[0], (((1,), (1,)), ((), ())),
                         preferred_element_type=jnp.float32)
    eo = eo + outb_ref[0]
    r = b * BT + lax.broadcasted_iota(jnp.int32, (BT, 1), 0)
    keep = (r >= offs_ref[e]) & (r < offs_ref[e + 1]) & (valid_ref[s] > 0)
    contrib = jnp.where(keep, eo, 0.0)

    @pl.when(first_ref[s] == 1)
    def _():
        ys_ref[...] = contrib

    @pl.when(first_ref[s] != 1)
    def _():
        ys_ref[...] = ys_ref[...] + contrib


def _ffn_call(bid, eid, first, valid, offs, xs, fc_w, fc_b, out_w, out_b):
    grid_spec = pltpu.PrefetchScalarGridSpec(
        num_scalar_prefetch=5,
        grid=(STEPS,),
        in_specs=[
            pl.BlockSpec((BT, H), lambda s, bid, eid, f, v, o: (bid[s], 0)),
            pl.BlockSpec((1, F2, H), lambda s, bid, eid, f, v, o: (eid[s], 0, 0)),
            pl.BlockSpec((1, 1, F2), lambda s, bid, eid, f, v, o: (eid[s], 0, 0)),
            pl.BlockSpec((1, H, F_DIM), lambda s, bid, eid, f, v, o: (eid[s], 0, 0)),
            pl.BlockSpec((1, 1, H), lambda s, bid, eid, f, v, o: (eid[s], 0, 0)),
        ],
        out_specs=pl.BlockSpec((BT, H), lambda s, bid, eid, f, v, o: (bid[s], 0)),
    )
    return pl.pallas_call(
        _ffn_body,
        grid_spec=grid_spec,
        out_shape=jax.ShapeDtypeStruct((T, H), jnp.float32),
    )(bid, eid, first, valid, offs,
      xs, fc_w, fc_b.reshape(E, 1, F2), out_w, out_b.reshape(E, 1, H))


def _step_tables(counts):
    offs = jnp.concatenate(
        [jnp.zeros((1,), jnp.int32), jnp.cumsum(counts, dtype=jnp.int32)])
    blk_start = offs[:E] // BT
    blk_end = jnp.where(counts > 0, (offs[1:] - 1) // BT, blk_start - 1)
    nblk = jnp.maximum(blk_end - blk_start + 1, 0)
    cum = jnp.cumsum(nblk)
    cum_excl = cum - nblk
    s = jnp.arange(STEPS, dtype=jnp.int32)
    eid = jnp.searchsorted(cum, s, side="right").astype(jnp.int32)
    valid = (s < cum[E - 1]).astype(jnp.int32)
    eid = jnp.minimum(eid, E - 1)
    bid = blk_start[eid] + (s - cum_excl[eid])
    bid = jnp.where(valid == 1, bid, NB - 1).astype(jnp.int32)
    prev = jnp.concatenate([jnp.full((1,), -1, jnp.int32), bid[:-1]])
    first = (bid != prev).astype(jnp.int32)
    offs16 = jnp.pad(offs, (0, 16 - (E + 1)))
    return bid, eid, first, valid, offs16


def kernel(x, gate_w, gate_b, fc_w, fc_b, out_w, out_b):
    pos2, counts2, loss11 = _router_call(x, gate_w, gate_b)
    counts = counts2.reshape(E)
    bid, eid, first, valid, offs16 = _step_tables(counts)
    pos = pos2.reshape(T)
    xs = _get_dispatch()(x, pos)
    ys = _ffn_call(bid, eid, first, valid, offs16, xs, fc_w, fc_b, out_w, out_b)
    out = _get_combine()(ys, pos)
    return out, loss11.reshape(())


# speedup vs baseline: 2.9695x; 1.0022x over previous
import functools

import jax
import jax.numpy as jnp
from jax import lax
from jax.experimental import pallas as pl
from jax.experimental.pallas import tpu as pltpu
from jax.experimental.pallas import tpu_sc as plsc

H = 768
F_DIM = 768
F2 = 2 * F_DIM
E = 8
T = 2048
BT = 256
NB = T // BT
STEPS = NB + E - 1
NC = 2
NS = 16
NW = NC * NS
CHUNK = T // NW


def _erf(v):
    p = 0.3275911
    a1, a2, a3, a4, a5 = (0.254829592, -0.284496736, 1.421413741,
                          -1.453152027, 1.061405429)
    sg = jnp.sign(v)
    av = jnp.abs(v)
    t = 1.0 / (1.0 + p * av)
    poly = ((((a5 * t + a4) * t + a3) * t + a2) * t + a1) * t
    return sg * (1.0 - poly * jnp.exp(-av * av))


def _gelu(v):
    return 0.5 * v * (1.0 + _erf(v * 0.7071067811865476))


def _router_body(gate_w_ref, gate_b_ref, x_ref, pos_ref, counts_ref,
                 loss_ref, cnt_scr, offs_scr):
    s = pl.program_id(0)

    @pl.when(s == 0)
    def _():
        cnt_scr[...] = jnp.zeros_like(cnt_scr)
        offs_scr[...] = jnp.zeros_like(offs_scr)

    @pl.when(s == NB)
    def _():
        def _sh(u, k):
            return jnp.concatenate(
                [jnp.zeros((1, k), jnp.float32), u[:, :E - k]], axis=1)

        t = cnt_scr[...]
        t = t + _sh(t, 1)
        t = t + _sh(t, 2)
        t = t + _sh(t, 4)
        offs_scr[...] = _sh(t, 1)
        cnt_scr[...] = jnp.zeros_like(cnt_scr)

    x = x_ref[...]
    raw = lax.dot_general(x, gate_w_ref[...], (((1,), (1,)), ((), ())),
                          preferred_element_type=jnp.float32)
    raw = raw + gate_b_ref[...]
    eids = lax.broadcasted_iota(jnp.int32, (BT, E), 1)
    m = jnp.max(raw, axis=1, keepdims=True)
    idx = jnp.min(jnp.where(raw == m, eids, E), axis=1)
    onehot = (eids == idx[:, None]).astype(jnp.bfloat16)

    rit = lax.broadcasted_iota(jnp.int32, (BT, BT), 0)
    cit = lax.broadcasted_iota(jnp.int32, (BT, BT), 1)
    ltri = (rit > cit).astype(jnp.bfloat16)
    within = lax.dot_general(ltri, onehot, (((1,), (0,)), ((), ())),
                             preferred_element_type=jnp.float32)
    rank_mat = within + cnt_scr[...]

    @pl.when(s >= NB)
    def _():
        pos_mat = rank_mat + offs_scr[...]
        pos = jnp.sum(jnp.where(eids == idx[:, None], pos_mat, 0.0), axis=1)
        pos_ref[...] = pos.astype(jnp.int32)[:, None]

    newcnt = cnt_scr[...] + jnp.sum(
        onehot.astype(jnp.float32), axis=0, keepdims=True)
    cnt_scr[...] = newcnt

    @pl.when(s == 2 * NB - 1)
    def _():
        counts_ref[...] = newcnt.astype(jnp.int32)
        usage = (newcnt > 0.0).astype(jnp.float32)
        loss_ref[...] = (jnp.sum((usage - 1.0 / E) ** 2) + 1e-8).reshape(1, 1)


def _router_call(x, gate_w, gate_b):
    return pl.pallas_call(
        _router_body,
        grid=(2 * NB,),
        in_specs=[
            pl.BlockSpec((E, H), lambda s: (0, 0)),
            pl.BlockSpec((1, E), lambda s: (0, 0)),
            pl.BlockSpec((BT, H), lambda s: (s % NB, 0)),
        ],
        out_specs=[
            pl.BlockSpec((BT, 1), lambda s: (s % NB, 0)),
            pl.BlockSpec((1, E), lambda s: (0, 0)),
            pl.BlockSpec((1, 1), lambda s: (0, 0)),
        ],
        out_shape=[
            jax.ShapeDtypeStruct((T, 1), jnp.int32),
            jax.ShapeDtypeStruct((1, E), jnp.int32),
            jax.ShapeDtypeStruct((1, 1), jnp.float32),
        ],
        scratch_shapes=[pltpu.VMEM((1, E), jnp.float32),
                        pltpu.VMEM((1, E), jnp.float32)],
    )(gate_w, gate_b.reshape(1, E), x)


@functools.cache
def _get_dispatch():
    mesh = plsc.VectorSubcoreMesh(core_axis_name="c", subcore_axis_name="s")

    @functools.partial(
        pl.kernel,
        mesh=mesh,
        out_type=jax.ShapeDtypeStruct((T, H), jnp.float32),
        scratch_types=[
            pltpu.VMEM((CHUNK,), jnp.int32),
            pltpu.VMEM((CHUNK, H), jnp.float32),
            pltpu.SemaphoreType.DMA,
        ],
    )
    def _dispatch(x_hbm, pos_hbm, xs_hbm, pos_v, x_v, sem):
        wid = lax.axis_index("s") * NC + lax.axis_index("c")
        base = wid * CHUNK
        pltpu.sync_copy(pos_hbm.at[pl.ds(base, CHUNK)], pos_v)
        pltpu.sync_copy(x_hbm.at[pl.ds(base, CHUNK)], x_v)
        pltpu.async_copy(x_v, xs_hbm.at[pos_v], sem).wait()

    return _dispatch


@functools.cache
def _get_combine():
    mesh = plsc.VectorSubcoreMesh(core_axis_name="c", subcore_axis_name="s")

    @functools.partial(
        pl.kernel,
        mesh=mesh,
        out_type=jax.ShapeDtypeStruct((T, H), jnp.float32),
        scratch_types=[
            pltpu.VMEM((CHUNK,), jnp.int32),
            pltpu.VMEM((CHUNK, H), jnp.float32),
            pltpu.SemaphoreType.DMA,
        ],
    )
    def _combine(ys_hbm, pos_hbm, out_hbm, pos_v, y_v, sem):
        wid = lax.axis_index("s") * NC + lax.axis_index("c")
        base = wid * CHUNK
        pltpu.sync_copy(pos_hbm.at[pl.ds(base, CHUNK)], pos_v)
        pltpu.async_copy(ys_hbm.at[pos_v], y_v, sem).wait()
        pltpu.sync_copy(y_v, out_hbm.at[pl.ds(base, CHUNK)])

    return _combine


def _ffn_body(bid_ref, eid_ref, first_ref, valid_ref, offs_ref,
              xs_ref, fcw_ref, fcb_ref, outw_ref, outb_ref, ys_ref):
    s = pl.program_id(0)
    e = eid_ref[s]
    b = bid_ref[s]
    x = xs_ref[...].astype(jnp.bfloat16)
    h = lax.dot_general(x, fcw_ref[0].astype(jnp.bfloat16),
                        (((1,), (1,)), ((), ())),
                        preferred_element_type=jnp.float32)
    h = h + fcb_ref[0]
    g = h[:, :F_DIM] * _gelu(h[:, F_DIM:])
    eo = lax.dot_general(g.astype(jnp.bfloat16),
                         outw_ref[0].astype(jnp.bfloat16),
                         (((1,), (1,)), ((), ())),
                         preferred_element_type=jnp.float32)
    eo = eo + outb_ref[0]
    r = b * BT + lax.broadcasted_iota(jnp.int32, (BT, 1), 0)
    keep = (r >= offs_ref[e]) & (r < offs_ref[e + 1]) & (valid_ref[s] > 0)
    contrib = jnp.where(keep, eo, 0.0)

    @pl.when(first_ref[s] == 1)
    def _():
        ys_ref[...] = contrib

    @pl.when(first_ref[s] != 1)
    def _():
        ys_ref[...] = ys_ref[...] + contrib


def _ffn_call(bid, eid, first, valid, offs, xs, fc_w, fc_b, out_w, out_b):
    grid_spec = pltpu.PrefetchScalarGridSpec(
        num_scalar_prefetch=5,
        grid=(STEPS,),
        in_specs=[
            pl.BlockSpec((BT, H), lambda s, bid, eid, f, v, o: (bid[s], 0)),
            pl.BlockSpec((1, F2, H), lambda s, bid, eid, f, v, o: (eid[s], 0, 0)),
            pl.BlockSpec((1, 1, F2), lambda s, bid, eid, f, v, o: (eid[s], 0, 0)),
            pl.BlockSpec((1, H, F_DIM), lambda s, bid, eid, f, v, o: (eid[s], 0, 0)),
            pl.BlockSpec((1, 1, H), lambda s, bid, eid, f, v, o: (eid[s], 0, 0)),
        ],
        out_specs=pl.BlockSpec((BT, H), lambda s, bid, eid, f, v, o: (bid[s], 0)),
    )
    return pl.pallas_call(
        _ffn_body,
        grid_spec=grid_spec,
        out_shape=jax.ShapeDtypeStruct((T, H), jnp.float32),
    )(bid, eid, first, valid, offs,
      xs, fc_w, fc_b.reshape(E, 1, F2), out_w, out_b.reshape(E, 1, H))


def _step_tables(counts):
    offs = jnp.concatenate(
        [jnp.zeros((1,), jnp.int32), jnp.cumsum(counts, dtype=jnp.int32)])
    blk_start = offs[:E] // BT
    blk_end = jnp.where(counts > 0, (offs[1:] - 1) // BT, blk_start - 1)
    nblk = jnp.maximum(blk_end - blk_start + 1, 0)
    cum = jnp.cumsum(nblk)
    cum_excl = cum - nblk
    s = jnp.arange(STEPS, dtype=jnp.int32)
    eid = jnp.searchsorted(cum, s, side="right").astype(jnp.int32)
    valid = (s < cum[E - 1]).astype(jnp.int32)
    eid = jnp.minimum(eid, E - 1)
    bid = blk_start[eid] + (s - cum_excl[eid])
    bid = jnp.where(valid == 1, bid, NB - 1).astype(jnp.int32)
    prev = jnp.concatenate([jnp.full((1,), -1, jnp.int32), bid[:-1]])
    first = (bid != prev).astype(jnp.int32)
    offs16 = jnp.pad(offs, (0, 16 - (E + 1)))
    return bid, eid, first, valid, offs16


def kernel(x, gate_w, gate_b, fc_w, fc_b, out_w, out_b):
    pos2, counts2, loss11 = _router_call(x, gate_w, gate_b)
    counts = counts2.reshape(E)
    bid, eid, first, valid, offs16 = _step_tables(counts)
    pos = pos2.reshape(T)
    xs = _get_dispatch()(x, pos)
    ys = _ffn_call(bid, eid, first, valid, offs16, xs, fc_w, fc_b, out_w, out_b)
    out = _get_combine()(ys, pos)
    return out, loss11.reshape(())


# speedup vs baseline: 3.2537x; 1.0957x over previous
import functools

import jax
import jax.numpy as jnp
from jax import lax
from jax.experimental import pallas as pl
from jax.experimental.pallas import tpu as pltpu
from jax.experimental.pallas import tpu_sc as plsc

H = 768
F_DIM = 768
F2 = 2 * F_DIM
E = 8
T = 2048
BT = 256
NB = T // BT
STEPS = NB + E - 1
NC = 2
NS = 16
NW = NC * NS
CHUNK = T // NW


def _erf(v):
    p = 0.3275911
    a1, a2, a3, a4, a5 = (0.254829592, -0.284496736, 1.421413741,
                          -1.453152027, 1.061405429)
    sg = jnp.sign(v)
    av = jnp.abs(v)
    t = 1.0 / (1.0 + p * av)
    poly = ((((a5 * t + a4) * t + a3) * t + a2) * t + a1) * t
    return sg * (1.0 - poly * jnp.exp(-av * av))


def _gelu(v):
    return 0.5 * v * (1.0 + _erf(v * 0.7071067811865476))


def _incl_scan16(v):
    def _sh(u, k):
        return jnp.concatenate(
            [jnp.zeros((1, k), jnp.float32), u[:, :16 - k]], axis=1)

    t = v + _sh(v, 1)
    t = t + _sh(t, 2)
    t = t + _sh(t, 4)
    return t + _sh(t, 8)


def _lanes_to_sublanes(v):
    d = (lax.broadcasted_iota(jnp.int32, (E, E), 0)
         == lax.broadcasted_iota(jnp.int32, (E, E), 1))
    return jnp.sum(jnp.where(d, v, 0.0), axis=1, keepdims=True)


def _router_body(gate_w_ref, gate_b_ref, x_ref, pos_ref, loss_ref, tbl_ref,
                 idx_scr, rank_scr, cnt_scr):
    s = pl.program_id(0)

    @pl.when(s == 0)
    def _():
        cnt_scr[...] = jnp.zeros_like(cnt_scr)

    @pl.when(s < NB)
    def _():
        x = x_ref[...]
        raw = lax.dot_general(x, gate_w_ref[...], (((1,), (1,)), ((), ())),
                              preferred_element_type=jnp.float32)
        raw = raw + gate_b_ref[...]
        eids = lax.broadcasted_iota(jnp.int32, (BT, E), 1)
        m = jnp.max(raw, axis=1, keepdims=True)
        idx = jnp.min(jnp.where(raw == m, eids, E), axis=1)
        onehot = (eids == idx[:, None]).astype(jnp.bfloat16)

        rit = lax.broadcasted_iota(jnp.int32, (BT, BT), 0)
        cit = lax.broadcasted_iota(jnp.int32, (BT, BT), 1)
        ltri = (rit > cit).astype(jnp.bfloat16)
        within = lax.dot_general(ltri, onehot, (((1,), (0,)), ((), ())),
                                 preferred_element_type=jnp.float32)
        rank_mat = within + cnt_scr[...]
        rank = jnp.sum(jnp.where(eids == idx[:, None], rank_mat, 0.0), axis=1)

        idx_scr[pl.ds(s * BT, BT), :] = idx[:, None]
        rank_scr[pl.ds(s * BT, BT), :] = rank.astype(jnp.int32)[:, None]
        cnt_scr[...] = cnt_scr[...] + jnp.sum(
            onehot.astype(jnp.float32), axis=0, keepdims=True)

    @pl.when(s == NB)
    def _():
        cnt = cnt_scr[...]
        usage = (cnt > 0.0).astype(jnp.float32)
        loss_ref[...] = (jnp.sum((usage - 1.0 / E) ** 2) + 1e-8).reshape(1, 1)

        c16 = jnp.concatenate([cnt, jnp.zeros((1, 16 - E), jnp.float32)],
                              axis=1)
        incl = _incl_scan16(c16)
        offs_row = incl - c16

        idx_all = idx_scr[...]
        rank_all = rank_scr[...]
        eids_t = lax.broadcasted_iota(jnp.int32, (T, E), 1)
        offs_e = offs_row[:, :E]
        picked = jnp.sum(jnp.where(eids_t == idx_all, offs_e, 0.0),
                         axis=1, keepdims=True)
        pos_ref[...] = rank_all + picked.astype(jnp.int32)

        lane16 = lax.broadcasted_iota(jnp.int32, (1, 16), 1)
        offs_i = offs_row.astype(jnp.int32)
        c_i = c16.astype(jnp.int32)
        blk_start = lax.shift_right_logical(offs_i, 8)
        blk_end = jnp.where(c_i > 0,
                            lax.shift_right_logical(offs_i + c_i - 1, 8),
                            blk_start - 1)
        nblk = jnp.maximum(blk_end - blk_start + 1, 0)
        nblk = jnp.where(lane16 < E, nblk, 0).astype(jnp.float32)
        cumnb = _incl_scan16(nblk)
        cumnb_excl = cumnb - nblk
        tp = jnp.max(cumnb)

        cumnb_sub = _lanes_to_sublanes(cumnb[:, :E])
        base_sub = _lanes_to_sublanes(
            (blk_start.astype(jnp.float32) - cumnb_excl)[:, :E])
        s16 = lane16.astype(jnp.float32)
        eid = jnp.sum((cumnb_sub <= s16).astype(jnp.float32),
                      axis=0, keepdims=True)
        eid = jnp.minimum(eid, float(E - 1))
        erow = lax.broadcasted_iota(jnp.int32, (E, 16), 0).astype(jnp.float32)
        base = jnp.sum(jnp.where(erow == eid, base_sub, 0.0),
                       axis=0, keepdims=True)
        valid = (s16 < tp).astype(jnp.float32)
        bid = jnp.where(valid > 0, base + s16, float(NB - 1))
        prev = jnp.concatenate(
            [jnp.full((1, 1), -1.0, jnp.float32), bid[:, :15]], axis=1)
        first = ((lane16 == 0) | (bid != prev)).astype(jnp.float32)

        z = jnp.zeros((1, 16), jnp.float32)
        tbl = jnp.concatenate(
            [bid, eid, first, valid, offs_row, z, z, z], axis=0)
        tbl_ref[...] = tbl.astype(jnp.int32)


def _router_call(x, gate_w, gate_b):
    return pl.pallas_call(
        _router_body,
        grid=(NB + 1,),
        in_specs=[
            pl.BlockSpec((E, H), lambda s: (0, 0)),
            pl.BlockSpec((1, E), lambda s: (0, 0)),
            pl.BlockSpec((BT, H), lambda s: (s % NB, 0)),
        ],
        out_specs=[
            pl.BlockSpec((T, 1), lambda s: (0, 0)),
            pl.BlockSpec((1, 1), lambda s: (0, 0)),
            pl.BlockSpec((E, 16), lambda s: (0, 0)),
        ],
        out_shape=[
            jax.ShapeDtypeStruct((T, 1), jnp.int32),
            jax.ShapeDtypeStruct((1, 1), jnp.float32),
            jax.ShapeDtypeStruct((E, 16), jnp.int32),
        ],
        scratch_shapes=[pltpu.VMEM((T, 1), jnp.int32),
                        pltpu.VMEM((T, 1), jnp.int32),
                        pltpu.VMEM((1, E), jnp.float32)],
    )(gate_w, gate_b.reshape(1, E), x)


@functools.cache
def _get_dispatch():
    mesh = plsc.VectorSubcoreMesh(core_axis_name="c", subcore_axis_name="s")

    @functools.partial(
        pl.kernel,
        mesh=mesh,
        out_type=jax.ShapeDtypeStruct((T, H), jnp.float32),
        scratch_types=[
            pltpu.VMEM((CHUNK,), jnp.int32),
            pltpu.VMEM((CHUNK, H), jnp.float32),
            pltpu.SemaphoreType.DMA,
        ],
    )
    def _dispatch(x_hbm, pos_hbm, xs_hbm, pos_v, x_v, sem):
        wid = lax.axis_index("s") * NC + lax.axis_index("c")
        base = wid * CHUNK
        pltpu.sync_copy(pos_hbm.at[pl.ds(base, CHUNK)], pos_v)
        pltpu.sync_copy(x_hbm.at[pl.ds(base, CHUNK)], x_v)
        pltpu.async_copy(x_v, xs_hbm.at[pos_v], sem).wait()

    return _dispatch


@functools.cache
def _get_combine():
    mesh = plsc.VectorSubcoreMesh(core_axis_name="c", subcore_axis_name="s")

    @functools.partial(
        pl.kernel,
        mesh=mesh,
        out_type=jax.ShapeDtypeStruct((T, H), jnp.float32),
        scratch_types=[
            pltpu.VMEM((CHUNK,), jnp.int32),
            pltpu.VMEM((CHUNK, H), jnp.float32),
            pltpu.SemaphoreType.DMA,
        ],
    )
    def _combine(ys_hbm, pos_hbm, out_hbm, pos_v, y_v, sem):
        wid = lax.axis_index("s") * NC + lax.axis_index("c")
        base = wid * CHUNK
        pltpu.sync_copy(pos_hbm.at[pl.ds(base, CHUNK)], pos_v)
        pltpu.async_copy(ys_hbm.at[pos_v], y_v, sem).wait()
        pltpu.sync_copy(y_v, out_hbm.at[pl.ds(base, CHUNK)])

    return _combine


def _ffn_body(tbl_ref, xs_ref, fcw_ref, fcb_ref, outw_ref, outb_ref, ys_ref):
    s = pl.program_id(0)
    e = tbl_ref[1, s]
    b = tbl_ref[0, s]
    x = xs_ref[...].astype(jnp.bfloat16)
    h = lax.dot_general(x, fcw_ref[0].astype(jnp.bfloat16),
                        (((1,), (1,)), ((), ())),
                        preferred_element_type=jnp.float32)
    h = h + fcb_ref[0]
    g = h[:, :F_DIM] * _gelu(h[:, F_DIM:])
    eo = lax.dot_general(g.astype(jnp.bfloat16),
                         outw_ref[0].astype(jnp.bfloat16),
                         (((1,), (1,)), ((), ())),
                         preferred_element_type=jnp.float32)
    eo = eo + outb_ref[0]
    r = b * BT + lax.broadcasted_iota(jnp.int32, (BT, 1), 0)
    keep = (r >= tbl_ref[4, e]) & (r < tbl_ref[4, e + 1]) & (tbl_ref[3, s] > 0)
    contrib = jnp.where(keep, eo, 0.0)

    @pl.when(tbl_ref[2, s] == 1)
    def _():
        ys_ref[...] = contrib

    @pl.when(tbl_ref[2, s] != 1)
    def _():
        ys_ref[...] = ys_ref[...] + contrib


def _ffn_call(tbl, xs, fc_w, fc_b, out_w, out_b):
    grid_spec = pltpu.PrefetchScalarGridSpec(
        num_scalar_prefetch=1,
        grid=(STEPS,),
        in_specs=[
            pl.BlockSpec((BT, H), lambda s, tbl: (tbl[0, s], 0)),
            pl.BlockSpec((1, F2, H), lambda s, tbl: (tbl[1, s], 0, 0)),
            pl.BlockSpec((1, 1, F2), lambda s, tbl: (tbl[1, s], 0, 0)),
            pl.BlockSpec((1, H, F_DIM), lambda s, tbl: (tbl[1, s], 0, 0)),
            pl.BlockSpec((1, 1, H), lambda s, tbl: (tbl[1, s], 0, 0)),
        ],
        out_specs=pl.BlockSpec((BT, H), lambda s, tbl: (tbl[0, s], 0)),
    )
    return pl.pallas_call(
        _ffn_body,
        grid_spec=grid_spec,
        out_shape=jax.ShapeDtypeStruct((T, H), jnp.float32),
    )(tbl, xs, fc_w, fc_b.reshape(E, 1, F2), out_w, out_b.reshape(E, 1, H))


def kernel(x, gate_w, gate_b, fc_w, fc_b, out_w, out_b):
    pos2, loss11, tbl = _router_call(x, gate_w, gate_b)
    pos = pos2.reshape(T)
    xs = _get_dispatch()(x, pos)
    ys = _ffn_call(tbl, xs, fc_w, fc_b, out_w, out_b)
    out = _get_combine()(ys, pos)
    return out, loss11.reshape(())
